# baseline (speedup 1.0000x reference)
import functools

import jax
import jax.numpy as jnp
from jax import lax
from jax.experimental import pallas as pl
from jax.experimental.pallas import tpu as pltpu
from jax.experimental.pallas import tpu_sc as plsc

H = 5120
I = 20480
NE = 3
EPS = 1e-6
T = 2048
TM = 256
NT = T // TM + (NE - 1)
TPAD = NT * TM

SC_NC = 2
SC_NS = 16
NW = SC_NC * SC_NS
ROWS_PER_W = TPAD // NW
GCH = 8
NCH = ROWS_PER_W // GCH

NB_UP = 1024
KB_UP = 1280
NK_UP = H // KB_UP
NB_DN = 1024
KB_DN = 1280
NK_DN = I // KB_DN


def _sc_mesh():
    return plsc.VectorSubcoreMesh(
        core_axis_name="c", subcore_axis_name="s",
        num_cores=SC_NC, num_subcores=SC_NS)


def _wid():
    return lax.axis_index("s") * SC_NC + lax.axis_index("c")


def _gather_body(x_hbm, idx_hbm, out_hbm, idx_v, rows_v, sem):
    base = _wid() * ROWS_PER_W
    pltpu.sync_copy(idx_hbm.at[pl.ds(base, ROWS_PER_W)], idx_v)
    for j in range(NCH):
        pltpu.async_copy(
            x_hbm.at[idx_v.at[pl.ds(j * GCH, GCH)]], rows_v, sem).wait()
        pltpu.sync_copy(rows_v, out_hbm.at[pl.ds(base + j * GCH, GCH)])


@functools.cache
def _gather_kernel():
    return pl.kernel(
        _gather_body, mesh=_sc_mesh(),
        out_type=jax.ShapeDtypeStruct((TPAD, H), jnp.float32),
        scratch_types=[
            pltpu.VMEM((ROWS_PER_W,), jnp.int32),
            pltpu.VMEM((GCH, H), jnp.float32),
            pltpu.SemaphoreType.DMA,
        ],
    )


def _gather_rows(x, g):
    return _gather_kernel()(x, g)


def _scatter_body(rows_hbm, idx_hbm, out_hbm, idx_v, rows_v, sem):
    base = _wid() * ROWS_PER_W
    pltpu.sync_copy(idx_hbm.at[_wid()], idx_v)
    for j in range(NCH):
        pltpu.sync_copy(rows_hbm.at[pl.ds(base + j * GCH, GCH)], rows_v)
        pltpu.async_copy(rows_v, out_hbm.at[idx_v.at[j]], sem).wait()


@functools.cache
def _scatter_kernel():
    return pl.kernel(
        _scatter_body, mesh=_sc_mesh(),
        out_type=jax.ShapeDtypeStruct((T, H), jnp.float32),
        scratch_types=[
            pltpu.VMEM((NCH, GCH), jnp.int32),
            pltpu.VMEM((GCH, H), jnp.float32),
            pltpu.SemaphoreType.DMA,
        ],
    )


def _scatter_rows(rows, idx3):
    return _scatter_kernel()(rows, idx3)


def _norm_body(e_ref, x_ref, nw_ref, o_ref):
    del e_ref
    xb = x_ref[...]
    ms = jnp.mean(xb * xb, axis=1, keepdims=True)
    scaled = xb * lax.rsqrt(ms + EPS) * (nw_ref[0] + 1.0)
    o_ref[...] = scaled.astype(jnp.bfloat16)


def _rmsnorm(xg, norm_w2, e_of_t):
    return pl.pallas_call(
        _norm_body,
        grid_spec=pltpu.PrefetchScalarGridSpec(
            num_scalar_prefetch=1,
            grid=(NT,),
            in_specs=[
                pl.BlockSpec((TM, H), lambda t, e: (t, 0)),
                pl.BlockSpec((1, 1, H), lambda t, e: (e[t], 0, 0)),
            ],
            out_specs=pl.BlockSpec((TM, H), lambda t, e: (t, 0)),
        ),
        out_shape=jax.ShapeDtypeStruct((TPAD, H), jnp.bfloat16),
    )(e_of_t, xg, norm_w2)


def _up_body(e_ref, x_ref, w_ref, o_ref, acc_ref):
    del e_ref
    k = pl.program_id(1)
    t = pl.program_id(2)
    xb = x_ref[pl.ds(t * TM, TM), pl.ds(k * KB_UP, KB_UP)]
    part = lax.dot_general(
        xb.astype(jnp.float32), w_ref[0],
        (((1,), (1,)), ((), ())),
        preferred_element_type=jnp.float32)

    @pl.when(k == 0)
    def _():
        acc_ref[t] = part

    @pl.when(k > 0)
    def _():
        acc_ref[t] = acc_ref[t] + part

    @pl.when(k == NK_UP - 1)
    def _():
        g = jnp.minimum(acc_ref[t], 7.0)
        o_ref[pl.ds(t * TM, TM), :] = (
            g * jax.nn.sigmoid(1.702 * g)).astype(jnp.bfloat16)


def _up_proj(xn, w_up3, e_of_t):
    return pl.pallas_call(
        _up_body,
        grid_spec=pltpu.PrefetchScalarGridSpec(
            num_scalar_prefetch=1,
            grid=(I // NB_UP, NK_UP, NT),
            in_specs=[
                pl.BlockSpec((TPAD, H), lambda n, k, t, e: (0, 0)),
                pl.BlockSpec((1, NB_UP, KB_UP), lambda n, k, t, e: (e[t], n, k)),
            ],
            out_specs=pl.BlockSpec((TPAD, NB_UP), lambda n, k, t, e: (0, n)),
            scratch_shapes=[pltpu.VMEM((NT, TM, NB_UP), jnp.float32)],
        ),
        out_shape=jax.ShapeDtypeStruct((TPAD, I), jnp.bfloat16),
    )(e_of_t, xn, w_up3)


def _down_body(e_ref, x_ref, w_ref, o_ref, acc_ref):
    del e_ref
    k = pl.program_id(1)
    t = pl.program_id(2)
    part = lax.dot_general(
        x_ref[...].astype(jnp.float32), w_ref[0],
        (((1,), (1,)), ((), ())),
        preferred_element_type=jnp.float32)

    @pl.when(k == 0)
    def _():
        acc_ref[t] = part

    @pl.when(k > 0)
    def _():
        acc_ref[t] = acc_ref[t] + part

    @pl.when(k == NK_DN - 1)
    def _():
        o_ref[pl.ds(t * TM, TM), :] = acc_ref[t]


def _down_proj(ug, w_dn3, e_of_t):
    return pl.pallas_call(
        _down_body,
        grid_spec=pltpu.PrefetchScalarGridSpec(
            num_scalar_prefetch=1,
            grid=(H // NB_DN, NK_DN, NT),
            in_specs=[
                pl.BlockSpec((TM, KB_DN), lambda n, k, t, e: (t, k)),
                pl.BlockSpec((1, NB_DN, KB_DN), lambda n, k, t, e: (e[t], n, k)),
            ],
            out_specs=pl.BlockSpec((TPAD, NB_DN), lambda n, k, t, e: (0, n)),
            scratch_shapes=[pltpu.VMEM((NT, TM, NB_DN), jnp.float32)],
        ),
        out_shape=jax.ShapeDtypeStruct((TPAD, H), jnp.float32),
    )(e_of_t, ug, w_dn3)


def _routing(modality_ids):
    ids = modality_ids.astype(jnp.int32)
    sort_idx = jnp.argsort(ids).astype(jnp.int32)
    counts = jnp.sum(ids[None, :] == jnp.arange(NE, dtype=jnp.int32)[:, None],
                     axis=1).astype(jnp.int32)
    offs = jnp.concatenate(
        [jnp.zeros((1,), jnp.int32), jnp.cumsum(counts)[:-1].astype(jnp.int32)])
    tiles_e = (counts + TM - 1) // TM
    tend = jnp.cumsum(tiles_e).astype(jnp.int32)
    tstart = tend - tiles_e
    big = jnp.argmax(counts).astype(jnp.int32)

    t_arr = jnp.arange(NT, dtype=jnp.int32)
    raw = jnp.searchsorted(tend, t_arr, side="right").astype(jnp.int32)
    active_t = t_arr < tend[NE - 1]
    e_of_t = jnp.where(active_t, jnp.minimum(raw, NE - 1), big)

    s = jnp.arange(TPAD, dtype=jnp.int32)
    st = s // TM
    sr = s % TM
    se = e_of_t[st]
    local = (st - tstart[se]) * TM + sr
    ce = counts[se]
    act = active_t[st]
    pos = offs[se] + jnp.where(
        act & (local < ce), local, jnp.maximum(ce - 1, 0))
    g = sort_idx[pos]
    return g, e_of_t


def kernel(x, modality_ids, norm_w, W_up, W_down):
    g, e_of_t = _routing(modality_ids)
    xg = _gather_rows(x, g)
    xn = _rmsnorm(xg, norm_w.reshape(NE, 1, H), e_of_t)
    ug = _up_proj(xn, W_up.reshape(NE, I, H), e_of_t)
    og = _down_proj(ug, W_down.reshape(NE, H, I), e_of_t)
    out = _scatter_rows(og, g.reshape(NW, NCH, GCH))
    return out


# speedup vs baseline: 1.2744x; 1.2744x over previous
import functools

import jax
import jax.numpy as jnp
from jax import lax
from jax.experimental import pallas as pl
from jax.experimental.pallas import tpu as pltpu
from jax.experimental.pallas import tpu_sc as plsc

H = 5120
I = 20480
NE = 3
EPS = 1e-6
T = 2048
TM = 256
NT = T // TM + (NE - 1)
TPAD = NT * TM

SC_NC = 2
SC_NS = 16
NW = SC_NC * SC_NS
ROWS_PER_W = TPAD // NW
GCH = 8
NCH = ROWS_PER_W // GCH

NB_UP = 1024
NB_DN = 1280
KB_DN = 2560
NK_DN = I // KB_DN


def _sc_mesh():
    return plsc.VectorSubcoreMesh(
        core_axis_name="c", subcore_axis_name="s",
        num_cores=SC_NC, num_subcores=SC_NS)


def _wid():
    return lax.axis_index("s") * SC_NC + lax.axis_index("c")


def _gather_body(x_hbm, idx_hbm, out_hbm, idx_v, rows_v, sem):
    base = _wid() * ROWS_PER_W
    pltpu.sync_copy(idx_hbm.at[pl.ds(base, ROWS_PER_W)], idx_v)
    for j in range(NCH):
        pltpu.async_copy(
            x_hbm.at[idx_v.at[pl.ds(j * GCH, GCH)]], rows_v, sem).wait()
        pltpu.sync_copy(rows_v, out_hbm.at[pl.ds(base + j * GCH, GCH)])


@functools.cache
def _gather_kernel():
    return pl.kernel(
        _gather_body, mesh=_sc_mesh(),
        out_type=jax.ShapeDtypeStruct((TPAD, H), jnp.float32),
        scratch_types=[
            pltpu.VMEM((ROWS_PER_W,), jnp.int32),
            pltpu.VMEM((GCH, H), jnp.float32),
            pltpu.SemaphoreType.DMA,
        ],
    )


def _gather_rows(x, g):
    return _gather_kernel()(x, g)


def _scatter_body(rows_hbm, idx_hbm, out_hbm, idx_v, rows_v, sem):
    base = _wid() * ROWS_PER_W
    pltpu.sync_copy(idx_hbm.at[_wid()], idx_v)
    for j in range(NCH):
        pltpu.sync_copy(rows_hbm.at[pl.ds(base + j * GCH, GCH)], rows_v)
        pltpu.async_copy(rows_v, out_hbm.at[idx_v.at[j]], sem).wait()


@functools.cache
def _scatter_kernel():
    return pl.kernel(
        _scatter_body, mesh=_sc_mesh(),
        out_type=jax.ShapeDtypeStruct((T, H), jnp.float32),
        scratch_types=[
            pltpu.VMEM((NCH, GCH), jnp.int32),
            pltpu.VMEM((GCH, H), jnp.float32),
            pltpu.SemaphoreType.DMA,
        ],
    )


def _scatter_rows(rows, idx3):
    return _scatter_kernel()(rows, idx3)


def _norm_body(e_ref, x_ref, nw_ref, o_ref):
    del e_ref
    xb = x_ref[...]
    ms = jnp.mean(xb * xb, axis=1, keepdims=True)
    scaled = xb * lax.rsqrt(ms + EPS) * (nw_ref[0] + 1.0)
    o_ref[...] = scaled.astype(jnp.bfloat16)


def _rmsnorm(xg, norm_w2, e_of_t):
    return pl.pallas_call(
        _norm_body,
        grid_spec=pltpu.PrefetchScalarGridSpec(
            num_scalar_prefetch=1,
            grid=(NT,),
            in_specs=[
                pl.BlockSpec((TM, H), lambda t, e: (t, 0)),
                pl.BlockSpec((1, 1, H), lambda t, e: (e[t], 0, 0)),
            ],
            out_specs=pl.BlockSpec((TM, H), lambda t, e: (t, 0)),
        ),
        out_shape=jax.ShapeDtypeStruct((TPAD, H), jnp.bfloat16),
    )(e_of_t, xg, norm_w2)


def _up_body(e_ref, x_ref, w_ref, o_ref):
    del e_ref
    part = lax.dot_general(
        x_ref[...].astype(jnp.float32), w_ref[0],
        (((1,), (1,)), ((), ())),
        preferred_element_type=jnp.float32)
    g = jnp.minimum(part, 7.0)
    o_ref[...] = (g * jax.nn.sigmoid(1.702 * g)).astype(jnp.bfloat16)


def _up_proj(xn, w_up3, e_of_t):
    return pl.pallas_call(
        _up_body,
        grid_spec=pltpu.PrefetchScalarGridSpec(
            num_scalar_prefetch=1,
            grid=(I // NB_UP, NT),
            in_specs=[
                pl.BlockSpec((TM, H), lambda n, t, e: (t, 0)),
                pl.BlockSpec((1, NB_UP, H), lambda n, t, e: (e[t], n, 0)),
            ],
            out_specs=pl.BlockSpec((TM, NB_UP), lambda n, t, e: (t, n)),
        ),
        out_shape=jax.ShapeDtypeStruct((TPAD, I), jnp.bfloat16),
    )(e_of_t, xn, w_up3)


def _down_body(e_ref, x_ref, w_ref, o_ref):
    del e_ref
    k = pl.program_id(1)
    t = pl.program_id(2)
    part = lax.dot_general(
        x_ref[...].astype(jnp.float32), w_ref[0],
        (((1,), (1,)), ((), ())),
        preferred_element_type=jnp.float32)
    rows = pl.ds(t * TM, TM)

    @pl.when(k == 0)
    def _():
        o_ref[rows, :] = part

    @pl.when(k > 0)
    def _():
        o_ref[rows, :] = o_ref[rows, :] + part


def _down_proj(ug, w_dn3, e_of_t):
    return pl.pallas_call(
        _down_body,
        grid_spec=pltpu.PrefetchScalarGridSpec(
            num_scalar_prefetch=1,
            grid=(H // NB_DN, NK_DN, NT),
            in_specs=[
                pl.BlockSpec((TM, KB_DN), lambda n, k, t, e: (t, k)),
                pl.BlockSpec((1, NB_DN, KB_DN), lambda n, k, t, e: (e[t], n, k)),
            ],
            out_specs=pl.BlockSpec((TPAD, NB_DN), lambda n, k, t, e: (0, n)),
        ),
        out_shape=jax.ShapeDtypeStruct((TPAD, H), jnp.float32),
    )(e_of_t, ug, w_dn3)


def _routing(modality_ids):
    ids = modality_ids.astype(jnp.int32)
    sort_idx = jnp.argsort(ids).astype(jnp.int32)
    counts = jnp.sum(ids[None, :] == jnp.arange(NE, dtype=jnp.int32)[:, None],
                     axis=1).astype(jnp.int32)
    offs = jnp.concatenate(
        [jnp.zeros((1,), jnp.int32), jnp.cumsum(counts)[:-1].astype(jnp.int32)])
    tiles_e = (counts + TM - 1) // TM
    tend = jnp.cumsum(tiles_e).astype(jnp.int32)
    tstart = tend - tiles_e
    big = jnp.argmax(counts).astype(jnp.int32)

    t_arr = jnp.arange(NT, dtype=jnp.int32)
    raw = jnp.searchsorted(tend, t_arr, side="right").astype(jnp.int32)
    active_t = t_arr < tend[NE - 1]
    e_of_t = jnp.where(active_t, jnp.minimum(raw, NE - 1), big)

    s = jnp.arange(TPAD, dtype=jnp.int32)
    st = s // TM
    sr = s % TM
    se = e_of_t[st]
    local = (st - tstart[se]) * TM + sr
    ce = counts[se]
    act = active_t[st]
    pos = offs[se] + jnp.where(
        act & (local < ce), local, jnp.maximum(ce - 1, 0))
    g = sort_idx[pos]
    return g, e_of_t


def kernel(x, modality_ids, norm_w, W_up, W_down):
    g, e_of_t = _routing(modality_ids)
    xg = _gather_rows(x, g)
    xn = _rmsnorm(xg, norm_w.reshape(NE, 1, H), e_of_t)
    ug = _up_proj(xn, W_up.reshape(NE, I, H), e_of_t)
    og = _down_proj(ug, W_down.reshape(NE, H, I), e_of_t)
    out = _scatter_rows(og, g.reshape(NW, NCH, GCH))
    return out


# speedup vs baseline: 1.2791x; 1.0037x over previous
import functools

import jax
import jax.numpy as jnp
from jax import lax
from jax.experimental import pallas as pl
from jax.experimental.pallas import tpu as pltpu
from jax.experimental.pallas import tpu_sc as plsc

H = 5120
I = 20480
NE = 3
EPS = 1e-6
T = 2048
TM = 256
NT = T // TM + (NE - 1)
TPAD = NT * TM

SC_NC = 2
SC_NS = 16
NW = SC_NC * SC_NS
ROWS_PER_W = TPAD // NW
GCH = 8
NCH = ROWS_PER_W // GCH

NB_UP = 1024
NB_DN = 1280
KB_DN = 2560
NK_DN = I // KB_DN


def _sc_mesh():
    return plsc.VectorSubcoreMesh(
        core_axis_name="c", subcore_axis_name="s",
        num_cores=SC_NC, num_subcores=SC_NS)


def _wid():
    return lax.axis_index("s") * SC_NC + lax.axis_index("c")


def _gather_body(x_hbm, idx_hbm, out_hbm, idx_v, rows_v, sem):
    base = _wid() * ROWS_PER_W
    pltpu.sync_copy(idx_hbm.at[pl.ds(base, ROWS_PER_W)], idx_v)
    for j in range(NCH):
        pltpu.async_copy(
            x_hbm.at[idx_v.at[pl.ds(j * GCH, GCH)]], rows_v, sem).wait()
        pltpu.sync_copy(rows_v, out_hbm.at[pl.ds(base + j * GCH, GCH)])


@functools.cache
def _gather_kernel():
    return pl.kernel(
        _gather_body, mesh=_sc_mesh(),
        out_type=jax.ShapeDtypeStruct((TPAD, H), jnp.float32),
        scratch_types=[
            pltpu.VMEM((ROWS_PER_W,), jnp.int32),
            pltpu.VMEM((GCH, H), jnp.float32),
            pltpu.SemaphoreType.DMA,
        ],
    )


def _gather_rows(x, g):
    return _gather_kernel()(x, g)


def _scatter_body(rows_hbm, idx_hbm, out_hbm, idx_v, rows_v, sem):
    base = _wid() * ROWS_PER_W
    pltpu.sync_copy(idx_hbm.at[_wid()], idx_v)
    for j in range(NCH):
        pltpu.sync_copy(rows_hbm.at[pl.ds(base + j * GCH, GCH)], rows_v)
        pltpu.async_copy(rows_v, out_hbm.at[idx_v.at[j]], sem).wait()


@functools.cache
def _scatter_kernel():
    return pl.kernel(
        _scatter_body, mesh=_sc_mesh(),
        out_type=jax.ShapeDtypeStruct((T, H), jnp.float32),
        scratch_types=[
            pltpu.VMEM((NCH, GCH), jnp.int32),
            pltpu.VMEM((GCH, H), jnp.float32),
            pltpu.SemaphoreType.DMA,
        ],
    )


def _scatter_rows(rows, idx3):
    return _scatter_kernel()(rows, idx3)


def _norm_body(e_ref, x_ref, nw_ref, o_ref):
    del e_ref
    xb = x_ref[...]
    ms = jnp.mean(xb * xb, axis=1, keepdims=True)
    scaled = xb * lax.rsqrt(ms + EPS) * (nw_ref[0] + 1.0)
    o_ref[...] = scaled.astype(jnp.bfloat16)


def _rmsnorm(xg, norm_w2, e_of_t):
    return pl.pallas_call(
        _norm_body,
        grid_spec=pltpu.PrefetchScalarGridSpec(
            num_scalar_prefetch=1,
            grid=(NT,),
            in_specs=[
                pl.BlockSpec((TM, H), lambda t, e: (t, 0)),
                pl.BlockSpec((1, 1, H), lambda t, e: (e[t], 0, 0)),
            ],
            out_specs=pl.BlockSpec((TM, H), lambda t, e: (t, 0)),
        ),
        out_shape=jax.ShapeDtypeStruct((TPAD, H), jnp.bfloat16),
    )(e_of_t, xg, norm_w2)


def _up_body(e_ref, x_ref, w_ref, o_ref):
    del e_ref
    part = lax.dot_general(
        x_ref[...].astype(jnp.float32), w_ref[0],
        (((1,), (1,)), ((), ())),
        preferred_element_type=jnp.float32)
    g = jnp.minimum(part, 7.0)
    o_ref[...] = (g * jax.nn.sigmoid(1.702 * g)).astype(jnp.bfloat16)


def _snake(n, t):
    return jnp.where(n % 2 == 0, t, NT - 1 - t)


def _up_proj(xn, w_up3, e_of_t):
    return pl.pallas_call(
        _up_body,
        grid_spec=pltpu.PrefetchScalarGridSpec(
            num_scalar_prefetch=1,
            grid=(I // NB_UP, NT),
            in_specs=[
                pl.BlockSpec((TM, H), lambda n, t, e: (_snake(n, t), 0)),
                pl.BlockSpec(
                    (1, NB_UP, H), lambda n, t, e: (e[_snake(n, t)], n, 0)),
            ],
            out_specs=pl.BlockSpec(
                (TM, NB_UP), lambda n, t, e: (_snake(n, t), n)),
        ),
        out_shape=jax.ShapeDtypeStruct((TPAD, I), jnp.bfloat16),
    )(e_of_t, xn, w_up3)


def _down_body(e_ref, x_ref, w_ref, o_ref):
    del e_ref
    n = pl.program_id(0)
    k = pl.program_id(1)
    t = pl.program_id(2)
    te = _snake(n * NK_DN + k, t)
    part = lax.dot_general(
        x_ref[...].astype(jnp.float32), w_ref[0],
        (((1,), (1,)), ((), ())),
        preferred_element_type=jnp.float32)
    rows = pl.ds(te * TM, TM)

    @pl.when(k == 0)
    def _():
        o_ref[rows, :] = part

    @pl.when(k > 0)
    def _():
        o_ref[rows, :] = o_ref[rows, :] + part


def _down_proj(ug, w_dn3, e_of_t):
    return pl.pallas_call(
        _down_body,
        grid_spec=pltpu.PrefetchScalarGridSpec(
            num_scalar_prefetch=1,
            grid=(H // NB_DN, NK_DN, NT),
            in_specs=[
                pl.BlockSpec(
                    (TM, KB_DN),
                    lambda n, k, t, e: (_snake(n * NK_DN + k, t), k)),
                pl.BlockSpec(
                    (1, NB_DN, KB_DN),
                    lambda n, k, t, e: (e[_snake(n * NK_DN + k, t)], n, k)),
            ],
            out_specs=pl.BlockSpec((TPAD, NB_DN), lambda n, k, t, e: (0, n)),
        ),
        out_shape=jax.ShapeDtypeStruct((TPAD, H), jnp.float32),
    )(e_of_t, ug, w_dn3)


def _routing(modality_ids):
    ids = modality_ids.astype(jnp.int32)
    sort_idx = jnp.argsort(ids).astype(jnp.int32)
    counts = jnp.sum(ids[None, :] == jnp.arange(NE, dtype=jnp.int32)[:, None],
                     axis=1).astype(jnp.int32)
    offs = jnp.concatenate(
        [jnp.zeros((1,), jnp.int32), jnp.cumsum(counts)[:-1].astype(jnp.int32)])
    tiles_e = (counts + TM - 1) // TM
    tend = jnp.cumsum(tiles_e).astype(jnp.int32)
    tstart = tend - tiles_e
    big = jnp.argmax(counts).astype(jnp.int32)

    t_arr = jnp.arange(NT, dtype=jnp.int32)
    raw = jnp.searchsorted(tend, t_arr, side="right").astype(jnp.int32)
    active_t = t_arr < tend[NE - 1]
    e_of_t = jnp.where(active_t, jnp.minimum(raw, NE - 1), big)

    s = jnp.arange(TPAD, dtype=jnp.int32)
    st = s // TM
    sr = s % TM
    se = e_of_t[st]
    local = (st - tstart[se]) * TM + sr
    ce = counts[se]
    act = active_t[st]
    pos = offs[se] + jnp.where(
        act & (local < ce), local, jnp.maximum(ce - 1, 0))
    g = sort_idx[pos]
    return g, e_of_t


def kernel(x, modality_ids, norm_w, W_up, W_down):
    g, e_of_t = _routing(modality_ids)
    xg = _gather_rows(x, g)
    xn = _rmsnorm(xg, norm_w.reshape(NE, 1, H), e_of_t)
    ug = _up_proj(xn, W_up.reshape(NE, I, H), e_of_t)
    og = _down_proj(ug, W_down.reshape(NE, H, I), e_of_t)
    out = _scatter_rows(og, g.reshape(NW, NCH, GCH))
    return out
